# baseline (speedup 1.0000x reference)
import functools

import jax
import jax.numpy as jnp
from jax import lax
from jax.experimental import pallas as pl
from jax.experimental.pallas import tpu as pltpu
from jax.experimental.pallas import tpu_sc as plsc

N = 100000
B = 2
H = 360
W = 360
C = 256
D = 128
NH = 8
HD = 16
M = B * H * W
NW = 32
SEC = 8112
M32 = NW * SEC
RB = 312
NCH = SEC // RB

_mesh = plsc.VectorSubcoreMesh(core_axis_name="c", subcore_axis_name="s")


@functools.partial(
    pl.kernel,
    mesh=_mesh,
    out_type=jax.ShapeDtypeStruct((M32,), jnp.int32),
    scratch_types=[
        pltpu.VMEM((N,), jnp.int32),
        pltpu.VMEM((SEC,), jnp.int32),
    ],
    compiler_params=pltpu.CompilerParams(needs_layout_passes=False),
)
def _sc_winner(flat_hbm, win_hbm, flat_v, canvas_v):
    wid = lax.axis_index("s") * 2 + lax.axis_index("c")
    base = wid * SEC
    pltpu.sync_copy(flat_hbm, flat_v)

    neg = jnp.full((16,), -1, jnp.int32)

    def init(i, carry):
        canvas_v[pl.ds(i * 16, 16)] = neg
        return carry

    lax.fori_loop(0, SEC // 16, init, 0)

    iota = lax.iota(jnp.int32, 16)

    def body(j, carry):
        f = flat_v[pl.ds(j * 16, 16)]
        rel = f - base
        mask = (rel >= 0) & (rel < SEC)
        ids = iota + j * 16
        plsc.store_scatter(canvas_v, [rel], ids, mask=mask)
        return carry

    lax.fori_loop(0, N // 16, body, 0)
    pltpu.sync_copy(canvas_v, win_hbm.at[pl.ds(base, SEC)])


@functools.partial(
    pl.kernel,
    mesh=_mesh,
    out_type=jax.ShapeDtypeStruct((M32, D), jnp.float32),
    scratch_types=[
        pltpu.VMEM((SEC,), jnp.int32),
        pltpu.VMEM((RB, D), jnp.float32),
        pltpu.VMEM((RB, D), jnp.float32),
        pltpu.SemaphoreType.DMA,
        pltpu.SemaphoreType.DMA,
    ],
    compiler_params=pltpu.CompilerParams(needs_layout_passes=False),
)
def _sc_gather(win_hbm, vf_hbm, canvas_hbm, idx_v, buf_a, buf_b, sem_a, sem_b):
    wid = lax.axis_index("s") * 2 + lax.axis_index("c")
    base = wid * SEC
    pltpu.sync_copy(win_hbm.at[pl.ds(base, SEC)], idx_v)

    iota = lax.iota(jnp.int32, 16)

    def fix(j, carry):
        w = idx_v[pl.ds(j * 16, 16)]
        p = base + j * 16 + iota
        dummy = jnp.where(p >= N, p - N, p)
        dummy = jnp.where(dummy >= N, dummy - N, dummy)
        idx_v[pl.ds(j * 16, 16)] = jnp.where(w < 0, dummy, w)
        return carry

    lax.fori_loop(0, SEC // 16, fix, 0)

    def step(i, carry):
        off_a = (2 * i) * RB
        off_b = (2 * i + 1) * RB
        cp_a = pltpu.async_copy(vf_hbm.at[idx_v.at[pl.ds(off_a, RB)]], buf_a, sem_a)
        cp_b = pltpu.async_copy(vf_hbm.at[idx_v.at[pl.ds(off_b, RB)]], buf_b, sem_b)
        cp_a.wait()
        pltpu.sync_copy(buf_a, canvas_hbm.at[pl.ds(base + off_a, RB)])
        cp_b.wait()
        pltpu.sync_copy(buf_b, canvas_hbm.at[pl.ds(base + off_b, RB)])
        return carry

    lax.fori_loop(0, NCH // 2, step, 0)


TY = 8


def _bdot(a, b):
    return lax.dot_general(a, b, (((1,), (0,)), ((), ())),
                           preferred_element_type=jnp.float32)


def _dot_rt(a, b):
    return lax.dot_general(a, b, (((1,), (1,)), ((), ())),
                           preferred_element_type=jnp.float32)


def _tc_body(img_ref, cv_ref, win_ref, wq_ref, wk_ref, wv_ref, wo_ref,
             seg_ref, sog_ref, hs_ref, hst_ref, kb_ref, bq_ref, bv_ref,
             bo_ref, gse_ref, gso_ref, bse_ref, bso_ref, gr_ref, br_ref,
             o256_ref, o128_ref, out_ref):
    wq = wq_ref[...]
    wk = wk_ref[...]
    wv = wv_ref[...]
    wo = wo_ref[...]
    seg = seg_ref[...]
    sog = sog_ref[...]
    hs = hs_ref[...]
    hst = hst_ref[...]
    kb = kb_ref[...]
    bq_ = bq_ref[...]
    bv_ = bv_ref[...]
    bo_ = bo_ref[...]
    gse = gse_ref[...]
    gso = gso_ref[...]
    bse = bse_ref[...]
    bso = bso_ref[...]
    gr = gr_ref[...]
    br = br_ref[...]
    o256 = o256_ref[...]
    o128 = o128_ref[...]

    P = TY * W
    x = img_ref[0].reshape(C, P)
    mu = o256 @ x
    ss = o256 @ (x * x)
    rstd = lax.rsqrt(ss - mu * mu + 1e-5)

    xb = x.astype(jnp.bfloat16)
    s0 = jnp.transpose(cv_ref[...])
    s1 = (_bdot(seg, xb) - gse * mu) * rstd + bse
    s2 = (_bdot(sog, xb) - gso * mu) * rstd + bso
    src = jnp.maximum(s0, jnp.maximum(s1, s2))

    q = _bdot(wq, src.astype(jnp.bfloat16)) + bq_

    slots = (s0, s1, s2)
    logits = []
    vals = []
    for j in range(3):
        sb = slots[j].astype(jnp.bfloat16)
        k = _bdot(wk, sb) + kb[:, j:j + 1]
        v = _bdot(wv, sb) + bv_
        logits.append((hs @ (q * k)) * 0.25)
        vals.append(v)

    mx = jnp.maximum(logits[0], jnp.maximum(logits[1], logits[2]))
    e0 = jnp.exp(logits[0] - mx)
    e1 = jnp.exp(logits[1] - mx)
    e2 = jnp.exp(logits[2] - mx)
    rz = 1.0 / (e0 + e1 + e2)
    ctx = (hst @ (e0 * rz)) * vals[0]
    ctx = ctx + (hst @ (e1 * rz)) * vals[1]
    ctx = ctx + (hst @ (e2 * rz)) * vals[2]

    o = _bdot(wo, ctx.astype(jnp.bfloat16)) + bo_
    y = o + src
    mu2 = o128 @ y
    ss2 = o128 @ (y * y)
    y = (y - mu2) * lax.rsqrt(ss2 - mu2 * mu2 + 1e-5) * gr + br

    mask = win_ref[0] >= 0
    y = jnp.where(mask, y, 0.0)
    out_ref[0] = y.reshape(D, TY, W)


def _full(shape):
    rank = len(shape)
    return pl.BlockSpec(shape, lambda b, h, _r=rank: (0,) * _r)


def kernel(voxel_features, voxel_coords, spatial_features_img, ln_img_g,
           ln_img_b, pos_emb, Wq, bq, Wk, bk, Wv, bv, Wo, bo, ln_red_g,
           ln_red_b):
    flat = ((voxel_coords[:, 0] * H + voxel_coords[:, 2]) * W
            + voxel_coords[:, 3]).astype(jnp.int32)

    winner = _sc_winner(flat)
    canvas = _sc_gather(winner, voxel_features)

    NR = H // TY
    P = TY * W
    img4 = spatial_features_img
    win3 = winner[:M].reshape(B * NR, 1, P)

    ar = jnp.arange
    f32 = jnp.float32
    se = (ar(C)[None, :] == 2 * ar(D)[:, None]).astype(f32)
    so = (ar(C)[None, :] == 2 * ar(D)[:, None] + 1).astype(f32)
    hs = (ar(D)[None, :] // HD == ar(NH)[:, None]).astype(f32)
    hst = hs.T
    kb = (pos_emb @ Wk.T + bk).T
    seg = se * ln_img_g[None, :]
    sog = so * ln_img_g[None, :]
    gse = (se @ ln_img_g)[:, None]
    gso = (so @ ln_img_g)[:, None]
    bse = (se @ ln_img_b)[:, None]
    bso = (so @ ln_img_b)[:, None]
    o256 = jnp.full((1, C), 1.0 / C, f32)
    o128 = jnp.full((1, D), 1.0 / D, f32)

    out = pl.pallas_call(
        _tc_body,
        grid=(B, NR),
        in_specs=[
            pl.BlockSpec((1, C, TY, W), lambda b, h: (b, 0, h, 0)),
            pl.BlockSpec((P, D), lambda b, h: (b * (H // TY) + h, 0)),
            pl.BlockSpec((1, 1, P), lambda b, h: (b * (H // TY) + h, 0, 0)),
            _full((D, D)), _full((D, D)), _full((D, D)), _full((D, D)),
            _full((D, C)), _full((D, C)), _full((NH, D)), _full((D, NH)),
            _full((D, 3)), _full((D, 1)), _full((D, 1)), _full((D, 1)),
            _full((D, 1)), _full((D, 1)), _full((D, 1)), _full((D, 1)),
            _full((D, 1)), _full((D, 1)), _full((1, C)), _full((1, D)),
        ],
        out_specs=pl.BlockSpec((1, D, TY, W), lambda b, h: (b, 0, h, 0)),
        out_shape=jax.ShapeDtypeStruct((B, D, H, W), jnp.float32),
    )(img4, canvas, win3,
      Wq.astype(jnp.bfloat16), Wk.astype(jnp.bfloat16),
      Wv.astype(jnp.bfloat16), Wo.astype(jnp.bfloat16),
      seg.astype(jnp.bfloat16), sog.astype(jnp.bfloat16), hs, hst,
      kb, bq[:, None], bv[:, None], bo[:, None], gse, gso, bse, bso,
      ln_red_g[:, None], ln_red_b[:, None], o256, o128)
    return out


# speedup vs baseline: 1.0054x; 1.0054x over previous
import functools

import jax
import jax.numpy as jnp
from jax import lax
from jax.experimental import pallas as pl
from jax.experimental.pallas import tpu as pltpu
from jax.experimental.pallas import tpu_sc as plsc

N = 100000
B = 2
H = 360
W = 360
C = 256
D = 128
NH = 8
HD = 16
M = B * H * W
NW = 32
SEC = 8112
M32 = NW * SEC
RB = 312
NCH = SEC // RB

_mesh = plsc.VectorSubcoreMesh(core_axis_name="c", subcore_axis_name="s")


@functools.partial(
    pl.kernel,
    mesh=_mesh,
    out_type=jax.ShapeDtypeStruct((M32,), jnp.int32),
    scratch_types=[
        pltpu.VMEM((N,), jnp.int32),
        pltpu.VMEM((SEC,), jnp.int32),
    ],
    compiler_params=pltpu.CompilerParams(needs_layout_passes=False),
)
def _sc_winner(flat_hbm, win_hbm, flat_v, canvas_v):
    wid = lax.axis_index("s") * 2 + lax.axis_index("c")
    base = wid * SEC
    pltpu.sync_copy(flat_hbm, flat_v)

    neg = jnp.full((16,), -1, jnp.int32)

    def init(i, carry):
        canvas_v[pl.ds(i * 16, 16)] = neg
        return carry

    lax.fori_loop(0, SEC // 16, init, 0)

    iota = lax.iota(jnp.int32, 16)

    def body(j, carry):
        f = flat_v[pl.ds(j * 16, 16)]
        rel = f - base
        mask = (rel >= 0) & (rel < SEC)
        ids = iota + j * 16
        plsc.store_scatter(canvas_v, [rel], ids, mask=mask)
        return carry

    lax.fori_loop(0, N // 16, body, 0)
    pltpu.sync_copy(canvas_v, win_hbm.at[pl.ds(base, SEC)])


@functools.partial(
    pl.kernel,
    mesh=_mesh,
    out_type=jax.ShapeDtypeStruct((M32, D), jnp.float32),
    scratch_types=[
        pltpu.VMEM((SEC,), jnp.int32),
        pltpu.VMEM((RB, D), jnp.float32),
        pltpu.VMEM((RB, D), jnp.float32),
        pltpu.SemaphoreType.DMA,
        pltpu.SemaphoreType.DMA,
    ],
    compiler_params=pltpu.CompilerParams(needs_layout_passes=False),
)
def _sc_gather(win_hbm, vf_hbm, canvas_hbm, idx_v, buf_a, buf_b, sem_a, sem_b):
    wid = lax.axis_index("s") * 2 + lax.axis_index("c")
    base = wid * SEC
    pltpu.sync_copy(win_hbm.at[pl.ds(base, SEC)], idx_v)

    iota = lax.iota(jnp.int32, 16)

    def fix(j, carry):
        w = idx_v[pl.ds(j * 16, 16)]
        p = base + j * 16 + iota
        dummy = jnp.where(p >= N, p - N, p)
        dummy = jnp.where(dummy >= N, dummy - N, dummy)
        idx_v[pl.ds(j * 16, 16)] = jnp.where(w < 0, dummy, w)
        return carry

    lax.fori_loop(0, SEC // 16, fix, 0)

    def step(i, carry):
        off_a = (2 * i) * RB
        off_b = (2 * i + 1) * RB
        cp_a = pltpu.async_copy(vf_hbm.at[idx_v.at[pl.ds(off_a, RB)]], buf_a, sem_a)
        cp_b = pltpu.async_copy(vf_hbm.at[idx_v.at[pl.ds(off_b, RB)]], buf_b, sem_b)
        cp_a.wait()
        pltpu.sync_copy(buf_a, canvas_hbm.at[pl.ds(base + off_a, RB)])
        cp_b.wait()
        pltpu.sync_copy(buf_b, canvas_hbm.at[pl.ds(base + off_b, RB)])
        return carry

    lax.fori_loop(0, NCH // 2, step, 0)


TY = 8


def _tc_body(img_ref, cv_ref, win_ref, wq_ref, wk_ref, wv_ref, wo_ref,
             seg_ref, sog_ref, hs_ref, hst_ref, kb_ref, bq_ref, bv_ref,
             bo_ref, gse_ref, gso_ref, bse_ref, bso_ref, gr_ref, br_ref,
             o256_ref, o128_ref, out_ref):
    wq = wq_ref[...]
    wk = wk_ref[...]
    wv = wv_ref[...]
    wo = wo_ref[...]
    seg = seg_ref[...]
    sog = sog_ref[...]
    hs = hs_ref[...]
    hst = hst_ref[...]
    kb = kb_ref[...]
    bq_ = bq_ref[...]
    bv_ = bv_ref[...]
    bo_ = bo_ref[...]
    gse = gse_ref[...]
    gso = gso_ref[...]
    bse = bse_ref[...]
    bso = bso_ref[...]
    gr = gr_ref[...]
    br = br_ref[...]
    o256 = o256_ref[...]
    o128 = o128_ref[...]

    P = TY * W
    x = img_ref[0].reshape(C, P)
    mu = o256 @ x
    ss = o256 @ (x * x)
    rstd = lax.rsqrt(ss - mu * mu + 1e-5)

    s0 = jnp.transpose(cv_ref[...])
    s1 = (seg @ x - gse * mu) * rstd + bse
    s2 = (sog @ x - gso * mu) * rstd + bso
    src = jnp.maximum(s0, jnp.maximum(s1, s2))

    q = wq @ src + bq_

    slots = (s0, s1, s2)
    logits = []
    vals = []
    for j in range(3):
        k = wk @ slots[j] + kb[:, j:j + 1]
        v = wv @ slots[j] + bv_
        logits.append((hs @ (q * k)) * 0.25)
        vals.append(v)

    mx = jnp.maximum(logits[0], jnp.maximum(logits[1], logits[2]))
    e0 = jnp.exp(logits[0] - mx)
    e1 = jnp.exp(logits[1] - mx)
    e2 = jnp.exp(logits[2] - mx)
    rz = 1.0 / (e0 + e1 + e2)
    ctx = (hst @ (e0 * rz)) * vals[0]
    ctx = ctx + (hst @ (e1 * rz)) * vals[1]
    ctx = ctx + (hst @ (e2 * rz)) * vals[2]

    o = wo @ ctx + bo_
    y = o + src
    mu2 = o128 @ y
    ss2 = o128 @ (y * y)
    y = (y - mu2) * lax.rsqrt(ss2 - mu2 * mu2 + 1e-5) * gr + br

    mask = win_ref[0] >= 0
    y = jnp.where(mask, y, 0.0)
    out_ref[0] = y.reshape(D, TY, W)


def _full(shape):
    rank = len(shape)
    return pl.BlockSpec(shape, lambda b, h, _r=rank: (0,) * _r)


def kernel(voxel_features, voxel_coords, spatial_features_img, ln_img_g,
           ln_img_b, pos_emb, Wq, bq, Wk, bk, Wv, bv, Wo, bo, ln_red_g,
           ln_red_b):
    flat = ((voxel_coords[:, 0] * H + voxel_coords[:, 2]) * W
            + voxel_coords[:, 3]).astype(jnp.int32)

    winner = _sc_winner(flat)
    canvas = _sc_gather(winner, voxel_features)

    NR = H // TY
    P = TY * W
    img4 = spatial_features_img
    win3 = winner[:M].reshape(B * NR, 1, P)

    ar = jnp.arange
    f32 = jnp.float32
    se = (ar(C)[None, :] == 2 * ar(D)[:, None]).astype(f32)
    so = (ar(C)[None, :] == 2 * ar(D)[:, None] + 1).astype(f32)
    hs = (ar(D)[None, :] // HD == ar(NH)[:, None]).astype(f32)
    hst = hs.T
    kb = (pos_emb @ Wk.T + bk).T
    seg = se * ln_img_g[None, :]
    sog = so * ln_img_g[None, :]
    gse = (se @ ln_img_g)[:, None]
    gso = (so @ ln_img_g)[:, None]
    bse = (se @ ln_img_b)[:, None]
    bso = (so @ ln_img_b)[:, None]
    o256 = jnp.full((1, C), 1.0 / C, f32)
    o128 = jnp.full((1, D), 1.0 / D, f32)

    out = pl.pallas_call(
        _tc_body,
        grid=(B, NR),
        in_specs=[
            pl.BlockSpec((1, C, TY, W), lambda b, h: (b, 0, h, 0)),
            pl.BlockSpec((P, D), lambda b, h: (b * (H // TY) + h, 0)),
            pl.BlockSpec((1, 1, P), lambda b, h: (b * (H // TY) + h, 0, 0)),
            _full((D, D)), _full((D, D)), _full((D, D)), _full((D, D)),
            _full((D, C)), _full((D, C)), _full((NH, D)), _full((D, NH)),
            _full((D, 3)), _full((D, 1)), _full((D, 1)), _full((D, 1)),
            _full((D, 1)), _full((D, 1)), _full((D, 1)), _full((D, 1)),
            _full((D, 1)), _full((D, 1)), _full((1, C)), _full((1, D)),
        ],
        out_specs=pl.BlockSpec((1, D, TY, W), lambda b, h: (b, 0, h, 0)),
        out_shape=jax.ShapeDtypeStruct((B, D, H, W), jnp.float32),
    )(img4, canvas, win3, Wq, Wk, Wv, Wo, seg, sog, hs, hst,
      kb, bq[:, None], bv[:, None], bo[:, None], gse, gso, bse, bso,
      ln_red_g[:, None], ln_red_b[:, None], o256, o128)
    return out
